# baseline (speedup 1.0000x reference)
import functools

import jax
import jax.numpy as jnp
from jax import lax
from jax.experimental import pallas as pl
from jax.experimental.pallas import tpu as pltpu
from jax.experimental.pallas import tpu_sc as plsc

N = 10000
E = 320000
NC = 2
NS = 16
BLK = 128
G = 8
PBLK = 2560
EP = PBLK * BLK
NA = N + 8
GRP = 16
NG = N // GRP
NGJ = -(-NG // NS)
DEGW = 128
F32 = jnp.float32


def _mesh():
    return plsc.VectorSubcoreMesh(
        core_axis_name="c", subcore_axis_name="s",
        num_cores=NC, num_subcores=NS)


def _row_groups(sid, fn):
    def g(j, _):
        grp = sid + NS * j

        @pl.when(grp < NG)
        def _():
            fn(pl.multiple_of(grp * GRP, GRP))
        return 0
    lax.fori_loop(0, NGJ, g, 0)


def _fill_rows(ref, nrows, width, value):
    def outer(i, _):
        def inner(j, _):
            ref[i, pl.ds(j * 16, 16)] = jnp.full((16,), value, F32)
            return 0
        return lax.fori_loop(0, width // 16, inner, 0)
    lax.fori_loop(0, nrows, outer, 0)


def _span_gather_scatter(xc, src2, dst2, srcm, dstm, rows_v, acc,
                         gsem, b0, tot):
    MG = tot // G

    def mega(m, _):
        bo = b0 + m * G
        pltpu.sync_copy(src2.at[pl.ds(bo, G)], srcm)
        pltpu.sync_copy(dst2.at[pl.ds(bo, G)], dstm)
        descs = [pltpu.async_copy(xc.at[srcm.at[0]], rows_v.at[0], gsem)]
        for i in range(G):
            descs[i].wait()
            if i + 1 < G:
                descs.append(pltpu.async_copy(
                    xc.at[srcm.at[i + 1]], rows_v.at[(i + 1) & 1], gsem))
            pltpu.sync_copy(rows_v.at[i & 1], acc.at[dstm.at[i]], add=True)
        return 0
    lax.fori_loop(0, MG, mega, 0)


@functools.lru_cache(maxsize=None)
def _sc_agg_full(C, W):
    CPC = -(-C // NC)
    TOT = PBLK // NS
    scratch = [
        pltpu.VMEM((G, BLK), jnp.int32),
        pltpu.VMEM((G, BLK), jnp.int32),
        pltpu.VMEM((2, BLK, W), F32),
        pltpu.VMEM((GRP, W), F32),
        pltpu.VMEM_SHARED((NA, W), F32),
        pltpu.SemaphoreType.DMA,
    ]

    def body(xt, src2, dst2, out, srcm, dstm, rows_v, zrow_v, acc, gsem):
        cid = lax.axis_index("c")
        sid = lax.axis_index("s")
        _fill_rows(zrow_v, GRP, W, 0.0)

        def process_chunk(c):
            xc = xt.at[c]
            _row_groups(sid, lambda off: pltpu.sync_copy(
                zrow_v, acc.at[pl.ds(off, GRP)]))
            plsc.subcore_barrier()
            _span_gather_scatter(xc, src2, dst2, srcm, dstm, rows_v, acc,
                                 gsem, sid * TOT, TOT)
            plsc.subcore_barrier()
            _row_groups(sid, lambda off: pltpu.sync_copy(
                acc.at[pl.ds(off, GRP)], out.at[c, pl.ds(off, GRP)]))

        for cp in range(NC):
            @pl.when(cid == cp)
            def _(cp=cp):
                for k in range(CPC):
                    c = k * NC + cp
                    if c < C:
                        process_chunk(c)

    return pl.kernel(
        body,
        out_type=jax.ShapeDtypeStruct((C, N, W), F32),
        mesh=_mesh(),
        scratch_types=scratch,
    )


@functools.lru_cache(maxsize=None)
def _sc_agg_split(W):
    TOT = PBLK // (NC * NS)
    scratch = [
        pltpu.VMEM((G, BLK), jnp.int32),
        pltpu.VMEM((G, BLK), jnp.int32),
        pltpu.VMEM((2, BLK, W), F32),
        pltpu.VMEM((GRP, W), F32),
        pltpu.VMEM_SHARED((NA, W), F32),
        pltpu.SemaphoreType.DMA,
    ]

    def body(xt, src2, dst2, out, srcm, dstm, rows_v, zrow_v, acc, gsem):
        cid = lax.axis_index("c")
        sid = lax.axis_index("s")
        _fill_rows(zrow_v, GRP, W, 0.0)

        for cp in range(NC):
            @pl.when(cid == cp)
            def _(cp=cp):
                _row_groups(sid, lambda off: pltpu.sync_copy(
                    zrow_v, acc.at[pl.ds(off, GRP)]))
                plsc.subcore_barrier()
                _span_gather_scatter(
                    xt, src2, dst2, srcm, dstm, rows_v, acc, gsem,
                    (cp * NS + sid) * TOT, TOT)
                plsc.subcore_barrier()
                _row_groups(sid, lambda off: pltpu.sync_copy(
                    acc.at[pl.ds(off, GRP)], out.at[cp, pl.ds(off, GRP)]))

    return pl.kernel(
        body,
        out_type=jax.ShapeDtypeStruct((NC, N, W), F32),
        mesh=_mesh(),
        scratch_types=scratch,
    )


@functools.lru_cache(maxsize=None)
def _sc_deg():
    TOT = PBLK // (NC * NS)
    MG = TOT // G
    PAIRS = MG // 2
    scratch = [
        pltpu.VMEM((2 * G, BLK), jnp.int32),
        pltpu.VMEM((BLK, DEGW), F32),
        pltpu.VMEM((GRP, DEGW), F32),
        pltpu.VMEM_SHARED((NA, DEGW), F32),
        pltpu.SemaphoreType.DMA,
    ]

    def body(dst2, out, dstm, ones_v, zdeg_v, degacc, isem):
        cid = lax.axis_index("c")
        sid = lax.axis_index("s")
        _fill_rows(ones_v, BLK, DEGW, 1.0)
        _fill_rows(zdeg_v, GRP, DEGW, 0.0)

        def span_scatter_ones(b0):
            def idx_load(m, parity, sync):
                bo = b0 + m * G
                if sync:
                    pltpu.sync_copy(dst2.at[pl.ds(bo, G)],
                                    dstm.at[pl.ds(parity * G, G)])
                else:
                    pltpu.async_copy(dst2.at[pl.ds(bo, G)],
                                     dstm.at[pl.ds(parity * G, G)], isem)

            def idx_drain(parity):
                pltpu.make_async_copy(
                    dst2.at[pl.ds(0, G)],
                    dstm.at[pl.ds(parity * G, G)], isem).wait()

            def steps(parity):
                for i in range(G):
                    pltpu.sync_copy(
                        ones_v, degacc.at[dstm.at[parity * G + i]], add=True)

            idx_load(0, 0, sync=True)

            def pair(q, _):
                m0 = 2 * q
                last = q == PAIRS - 1
                idx_load(m0 + 1, 1, sync=False)
                steps(0)
                idx_drain(1)

                @pl.when(jnp.logical_not(last))
                def _():
                    idx_load(m0 + 2, 0, sync=False)
                steps(1)

                @pl.when(jnp.logical_not(last))
                def _():
                    idx_drain(0)
                return 0
            lax.fori_loop(0, PAIRS, pair, 0)

        for cp in range(NC):
            @pl.when(cid == cp)
            def _(cp=cp):
                _row_groups(sid, lambda off: pltpu.sync_copy(
                    zdeg_v, degacc.at[pl.ds(off, GRP)]))
                plsc.subcore_barrier()
                span_scatter_ones((cp * NS + sid) * TOT)
                plsc.subcore_barrier()
                _row_groups(sid, lambda off: pltpu.sync_copy(
                    degacc.at[pl.ds(off, GRP)], out.at[cp, pl.ds(off, GRP)]))

    return pl.kernel(
        body,
        out_type=jax.ShapeDtypeStruct((NC, N, DEGW), F32),
        mesh=_mesh(),
        scratch_types=scratch,
    )


@functools.lru_cache(maxsize=None)
def _tc_sage(C_in, dout, act, fuse_z, split_agg, BN=1000):
    C_out = dout // 128
    din = C_in * 128
    CA = NC if split_agg else C_in
    grid = (N // BN,)
    in_specs = [
        pl.BlockSpec((C_in, BN, 128), lambda i: (0, i, 0)),
        pl.BlockSpec((CA, BN, 128), lambda i: (0, i, 0)),
        pl.BlockSpec((NC, BN, DEGW), lambda i: (0, i, 0)),
        pl.BlockSpec((din, dout), lambda i: (0, 0)),
        pl.BlockSpec((din, dout), lambda i: (0, 0)),
        pl.BlockSpec((1, dout), lambda i: (0, 0)),
    ]
    out_shape = [jax.ShapeDtypeStruct((C_out, N, 128), F32)]
    out_specs = [pl.BlockSpec((C_out, BN, 128), lambda i: (0, i, 0))]
    if fuse_z:
        in_specs.append(pl.BlockSpec((dout, 128), lambda i: (0, 0)))
        out_shape.append(jax.ShapeDtypeStruct((N, 128), F32))
        out_specs.append(pl.BlockSpec((BN, 128), lambda i: (i, 0)))

    def body(h_ref, agg_ref, deg_ref, Ws_ref, Wn_ref, b_ref, *rest):
        if fuse_z:
            Wz_ref, out_ref, z_ref = rest
        else:
            (out_ref,) = rest
        deg = (deg_ref[0] + deg_ref[1])[:, 0:1]
        inv = 1.0 / jnp.maximum(deg, 1.0)
        acc = jnp.zeros((BN, dout), F32) + b_ref[...]
        for c in range(C_in):
            acc += jnp.dot(h_ref[c], Ws_ref[pl.ds(c * 128, 128), :],
                           preferred_element_type=F32)
            if not split_agg:
                acc += jnp.dot(agg_ref[c] * inv,
                               Wn_ref[pl.ds(c * 128, 128), :],
                               preferred_element_type=F32)
        if split_agg:
            a = (agg_ref[0] + agg_ref[1]) * inv
            acc += jnp.dot(a, Wn_ref[...], preferred_element_type=F32)
        if act:
            acc = jnp.maximum(acc, 0.0)
        for co in range(C_out):
            out_ref[co] = acc[:, co * 128:(co + 1) * 128]
        if fuse_z:
            z_ref[...] = jnp.dot(acc, Wz_ref[...], preferred_element_type=F32)

    return pl.pallas_call(
        body, grid=grid, in_specs=in_specs,
        out_specs=out_specs, out_shape=out_shape)


@functools.lru_cache(maxsize=None)
def _tc_final(dout=47, BN=1000):
    grid = (N // BN,)
    in_specs = [
        pl.BlockSpec((6, BN, 128), lambda i: (0, i, 0)),
        pl.BlockSpec((NC, BN, 128), lambda i: (0, i, 0)),
        pl.BlockSpec((NC, BN, DEGW), lambda i: (0, i, 0)),
        pl.BlockSpec((768, dout), lambda i: (0, 0)),
        pl.BlockSpec((1, dout), lambda i: (0, 0)),
    ]

    def body(h_ref, aggz_ref, deg_ref, Ws_ref, b_ref, out_ref):
        deg = (deg_ref[0] + deg_ref[1])[:, 0:1]
        inv = 1.0 / jnp.maximum(deg, 1.0)
        acc = jnp.zeros((BN, dout), F32) + b_ref[...]
        for c in range(6):
            acc += jnp.dot(h_ref[c], Ws_ref[pl.ds(c * 128, 128), :],
                           preferred_element_type=F32)
        az = aggz_ref[0] + aggz_ref[1]
        out_ref[...] = acc + az[:, 0:dout] * inv

    return pl.pallas_call(
        body, grid=grid, in_specs=in_specs,
        out_specs=pl.BlockSpec((BN, dout), lambda i: (i, 0)),
        out_shape=jax.ShapeDtypeStruct((N, dout), F32))


def kernel(x, edge_index,
           W_self_0, W_neigh_0, b_0, W_self_1, W_neigh_1, b_1,
           W_self_2, W_neigh_2, b_2, W_self_3, W_neigh_3, b_3,
           W_self_4, W_neigh_4, b_4):
    pad = EP - E
    src2 = jnp.concatenate(
        [edge_index[0], jnp.zeros((pad,), jnp.int32)]).reshape(PBLK, BLK)
    dst2 = jnp.concatenate(
        [edge_index[1], jnp.full((pad,), N, jnp.int32)]).reshape(PBLK, BLK)

    deg = _sc_deg()(dst2)
    agg = _sc_agg_split(128)(x, src2, dst2)
    h = _tc_sage(1, 768, True, False, True)(
        x.reshape(1, N, 128), agg, deg, W_self_0, W_neigh_0,
        b_0.reshape(1, 768))[0]

    for Ws, Wn, b in ((W_self_1, W_neigh_1, b_1), (W_self_2, W_neigh_2, b_2)):
        agg = _sc_agg_full(6, 128)(h, src2, dst2)
        h = _tc_sage(6, 768, True, False, False)(
            h, agg, deg, Ws, Wn, b.reshape(1, 768))[0]

    agg = _sc_agg_full(6, 128)(h, src2, dst2)
    Wn4 = jnp.pad(W_neigh_4, ((0, 0), (0, 128 - 47)))
    h, z = _tc_sage(6, 768, True, True, False)(
        h, agg, deg, W_self_3, W_neigh_3, b_3.reshape(1, 768), Wn4)

    aggz = _sc_agg_split(128)(z, src2, dst2)
    return _tc_final()(h, aggz, deg, W_self_4, b_4.reshape(1, 47))


# speedup vs baseline: 1.4129x; 1.4129x over previous
import functools

import jax
import jax.numpy as jnp
from jax import lax
from jax.experimental import pallas as pl
from jax.experimental.pallas import tpu as pltpu
from jax.experimental.pallas import tpu_sc as plsc

N = 10000
E = 320000
NC = 2
NS = 16
BLK = 128
G = 8
NBLK = E // BLK
PBLK = 2560
EP = PBLK * BLK
NA = N + 8
GRP = 16
NG = N // GRP
NGJ = -(-NG // NS)
DEGW = 128
F32 = jnp.float32


def _mesh():
    return plsc.VectorSubcoreMesh(
        core_axis_name="c", subcore_axis_name="s",
        num_cores=NC, num_subcores=NS)


def _row_groups(sid, fn):
    def g(j, _):
        grp = sid + NS * j

        @pl.when(grp < NG)
        def _():
            fn(pl.multiple_of(grp * GRP, GRP))
        return 0
    lax.fori_loop(0, NGJ, g, 0)


def _fill_rows(ref, nrows, width, value):
    def outer(i, _):
        def inner(j, _):
            ref[i, pl.ds(j * 16, 16)] = jnp.full((16,), value, F32)
            return 0
        return lax.fori_loop(0, width // 16, inner, 0)
    lax.fori_loop(0, nrows, outer, 0)


def _edge_serial(xc, src, dst, src_v, dst_v, rows_v, acc, gsem, sid,
                 stride_base, nj):
    def eb(j, _):
        blk = stride_base(j)

        @pl.when(blk < NBLK)
        def _():
            off = pl.multiple_of(blk * BLK, BLK)
            pltpu.sync_copy(src.at[pl.ds(off, BLK)], src_v)
            pltpu.sync_copy(dst.at[pl.ds(off, BLK)], dst_v)
            pltpu.async_copy(xc.at[src_v], rows_v, gsem).wait()
            pltpu.sync_copy(rows_v, acc.at[dst_v], add=True)
        return 0
    lax.fori_loop(0, nj, eb, 0)


@functools.lru_cache(maxsize=None)
def _sc_agg_full(C, W):
    CPC = -(-C // NC)
    NJ = -(-NBLK // NS)
    scratch = [
        pltpu.VMEM((BLK,), jnp.int32),
        pltpu.VMEM((BLK,), jnp.int32),
        pltpu.VMEM((BLK, W), F32),
        pltpu.VMEM((GRP, W), F32),
        pltpu.VMEM_SHARED((NA, W), F32),
        pltpu.SemaphoreType.DMA,
    ]

    def body(xt, src, dst, out, src_v, dst_v, rows_v, zrow_v, acc, gsem):
        cid = lax.axis_index("c")
        sid = lax.axis_index("s")
        _fill_rows(zrow_v, GRP, W, 0.0)

        def process_chunk(c):
            xc = xt.at[c]
            _row_groups(sid, lambda off: pltpu.sync_copy(
                zrow_v, acc.at[pl.ds(off, GRP)]))
            plsc.subcore_barrier()
            _edge_serial(xc, src, dst, src_v, dst_v, rows_v, acc, gsem,
                         sid, lambda j: sid + NS * j, NJ)
            plsc.subcore_barrier()
            _row_groups(sid, lambda off: pltpu.sync_copy(
                acc.at[pl.ds(off, GRP)], out.at[c, pl.ds(off, GRP)]))

        for cp in range(NC):
            @pl.when(cid == cp)
            def _(cp=cp):
                for k in range(CPC):
                    c = k * NC + cp
                    if c < C:
                        process_chunk(c)

    return pl.kernel(
        body,
        out_type=jax.ShapeDtypeStruct((C, N, W), F32),
        mesh=_mesh(),
        scratch_types=scratch,
    )


@functools.lru_cache(maxsize=None)
def _sc_agg_split(W):
    NJ = -(-NBLK // (NC * NS))
    scratch = [
        pltpu.VMEM((BLK,), jnp.int32),
        pltpu.VMEM((BLK,), jnp.int32),
        pltpu.VMEM((BLK, W), F32),
        pltpu.VMEM((GRP, W), F32),
        pltpu.VMEM_SHARED((NA, W), F32),
        pltpu.SemaphoreType.DMA,
    ]

    def body(xt, src, dst, out, src_v, dst_v, rows_v, zrow_v, acc, gsem):
        cid = lax.axis_index("c")
        sid = lax.axis_index("s")
        _fill_rows(zrow_v, GRP, W, 0.0)

        for cp in range(NC):
            @pl.when(cid == cp)
            def _(cp=cp):
                _row_groups(sid, lambda off: pltpu.sync_copy(
                    zrow_v, acc.at[pl.ds(off, GRP)]))
                plsc.subcore_barrier()
                _edge_serial(xt, src, dst, src_v, dst_v, rows_v, acc, gsem,
                             sid, lambda j: cp * NS + sid + NC * NS * j, NJ)
                plsc.subcore_barrier()
                _row_groups(sid, lambda off: pltpu.sync_copy(
                    acc.at[pl.ds(off, GRP)], out.at[cp, pl.ds(off, GRP)]))

    return pl.kernel(
        body,
        out_type=jax.ShapeDtypeStruct((NC, N, W), F32),
        mesh=_mesh(),
        scratch_types=scratch,
    )


@functools.lru_cache(maxsize=None)
def _sc_deg():
    TOT = PBLK // (NC * NS)
    MG = TOT // G
    PAIRS = MG // 2
    scratch = [
        pltpu.VMEM((2 * G, BLK), jnp.int32),
        pltpu.VMEM((BLK, DEGW), F32),
        pltpu.VMEM((GRP, DEGW), F32),
        pltpu.VMEM_SHARED((NA, DEGW), F32),
        pltpu.SemaphoreType.DMA,
    ]

    def body(dst2, out, dstm, ones_v, zdeg_v, degacc, isem):
        cid = lax.axis_index("c")
        sid = lax.axis_index("s")
        _fill_rows(ones_v, BLK, DEGW, 1.0)
        _fill_rows(zdeg_v, GRP, DEGW, 0.0)

        def span_scatter_ones(b0):
            def idx_load(m, parity, sync):
                bo = b0 + m * G
                if sync:
                    pltpu.sync_copy(dst2.at[pl.ds(bo, G)],
                                    dstm.at[pl.ds(parity * G, G)])
                else:
                    pltpu.async_copy(dst2.at[pl.ds(bo, G)],
                                     dstm.at[pl.ds(parity * G, G)], isem)

            def idx_drain(parity):
                pltpu.make_async_copy(
                    dst2.at[pl.ds(0, G)],
                    dstm.at[pl.ds(parity * G, G)], isem).wait()

            def steps(parity):
                for i in range(G):
                    pltpu.sync_copy(
                        ones_v, degacc.at[dstm.at[parity * G + i]], add=True)

            idx_load(0, 0, sync=True)

            def pair(q, _):
                m0 = 2 * q
                last = q == PAIRS - 1
                idx_load(m0 + 1, 1, sync=False)
                steps(0)
                idx_drain(1)

                @pl.when(jnp.logical_not(last))
                def _():
                    idx_load(m0 + 2, 0, sync=False)
                steps(1)

                @pl.when(jnp.logical_not(last))
                def _():
                    idx_drain(0)
                return 0
            lax.fori_loop(0, PAIRS, pair, 0)

        for cp in range(NC):
            @pl.when(cid == cp)
            def _(cp=cp):
                _row_groups(sid, lambda off: pltpu.sync_copy(
                    zdeg_v, degacc.at[pl.ds(off, GRP)]))
                plsc.subcore_barrier()
                span_scatter_ones((cp * NS + sid) * TOT)
                plsc.subcore_barrier()
                _row_groups(sid, lambda off: pltpu.sync_copy(
                    degacc.at[pl.ds(off, GRP)], out.at[cp, pl.ds(off, GRP)]))

    return pl.kernel(
        body,
        out_type=jax.ShapeDtypeStruct((NC, N, DEGW), F32),
        mesh=_mesh(),
        scratch_types=scratch,
    )


@functools.lru_cache(maxsize=None)
def _tc_sage(C_in, dout, act, fuse_z, split_agg, BN=1000):
    C_out = dout // 128
    din = C_in * 128
    CA = NC if split_agg else C_in
    grid = (N // BN,)
    in_specs = [
        pl.BlockSpec((C_in, BN, 128), lambda i: (0, i, 0)),
        pl.BlockSpec((CA, BN, 128), lambda i: (0, i, 0)),
        pl.BlockSpec((NC, BN, DEGW), lambda i: (0, i, 0)),
        pl.BlockSpec((din, dout), lambda i: (0, 0)),
        pl.BlockSpec((din, dout), lambda i: (0, 0)),
        pl.BlockSpec((1, dout), lambda i: (0, 0)),
    ]
    out_shape = [jax.ShapeDtypeStruct((C_out, N, 128), F32)]
    out_specs = [pl.BlockSpec((C_out, BN, 128), lambda i: (0, i, 0))]
    if fuse_z:
        in_specs.append(pl.BlockSpec((dout, 128), lambda i: (0, 0)))
        out_shape.append(jax.ShapeDtypeStruct((N, 128), F32))
        out_specs.append(pl.BlockSpec((BN, 128), lambda i: (i, 0)))

    def body(h_ref, agg_ref, deg_ref, Ws_ref, Wn_ref, b_ref, *rest):
        if fuse_z:
            Wz_ref, out_ref, z_ref = rest
        else:
            (out_ref,) = rest
        deg = (deg_ref[0] + deg_ref[1])[:, 0:1]
        inv = 1.0 / jnp.maximum(deg, 1.0)
        acc = jnp.zeros((BN, dout), F32) + b_ref[...]
        for c in range(C_in):
            acc += jnp.dot(h_ref[c], Ws_ref[pl.ds(c * 128, 128), :],
                           preferred_element_type=F32)
            if not split_agg:
                acc += jnp.dot(agg_ref[c] * inv,
                               Wn_ref[pl.ds(c * 128, 128), :],
                               preferred_element_type=F32)
        if split_agg:
            a = (agg_ref[0] + agg_ref[1]) * inv
            acc += jnp.dot(a, Wn_ref[...], preferred_element_type=F32)
        if act:
            acc = jnp.maximum(acc, 0.0)
        for co in range(C_out):
            out_ref[co] = acc[:, co * 128:(co + 1) * 128]
        if fuse_z:
            z_ref[...] = jnp.dot(acc, Wz_ref[...], preferred_element_type=F32)

    return pl.pallas_call(
        body, grid=grid, in_specs=in_specs,
        out_specs=out_specs, out_shape=out_shape)


@functools.lru_cache(maxsize=None)
def _tc_final(dout=47, BN=1000):
    grid = (N // BN,)
    in_specs = [
        pl.BlockSpec((6, BN, 128), lambda i: (0, i, 0)),
        pl.BlockSpec((NC, BN, 128), lambda i: (0, i, 0)),
        pl.BlockSpec((NC, BN, DEGW), lambda i: (0, i, 0)),
        pl.BlockSpec((768, dout), lambda i: (0, 0)),
        pl.BlockSpec((1, dout), lambda i: (0, 0)),
    ]

    def body(h_ref, aggz_ref, deg_ref, Ws_ref, b_ref, out_ref):
        deg = (deg_ref[0] + deg_ref[1])[:, 0:1]
        inv = 1.0 / jnp.maximum(deg, 1.0)
        acc = jnp.zeros((BN, dout), F32) + b_ref[...]
        for c in range(6):
            acc += jnp.dot(h_ref[c], Ws_ref[pl.ds(c * 128, 128), :],
                           preferred_element_type=F32)
        az = aggz_ref[0] + aggz_ref[1]
        out_ref[...] = acc + az[:, 0:dout] * inv

    return pl.pallas_call(
        body, grid=grid, in_specs=in_specs,
        out_specs=pl.BlockSpec((BN, dout), lambda i: (i, 0)),
        out_shape=jax.ShapeDtypeStruct((N, dout), F32))


def kernel(x, edge_index,
           W_self_0, W_neigh_0, b_0, W_self_1, W_neigh_1, b_1,
           W_self_2, W_neigh_2, b_2, W_self_3, W_neigh_3, b_3,
           W_self_4, W_neigh_4, b_4):
    src = edge_index[0]
    dst = edge_index[1]
    pad = EP - E
    dst2 = jnp.concatenate(
        [dst, jnp.full((pad,), N, jnp.int32)]).reshape(PBLK, BLK)

    deg = _sc_deg()(dst2)
    agg = _sc_agg_split(128)(x, src, dst)
    h = _tc_sage(1, 768, True, False, True)(
        x.reshape(1, N, 128), agg, deg, W_self_0, W_neigh_0,
        b_0.reshape(1, 768))[0]

    for Ws, Wn, b in ((W_self_1, W_neigh_1, b_1), (W_self_2, W_neigh_2, b_2)):
        agg = _sc_agg_full(6, 128)(h, src, dst)
        h = _tc_sage(6, 768, True, False, False)(
            h, agg, deg, Ws, Wn, b.reshape(1, 768))[0]

    agg = _sc_agg_full(6, 128)(h, src, dst)
    Wn4 = jnp.pad(W_neigh_4, ((0, 0), (0, 128 - 47)))
    h, z = _tc_sage(6, 768, True, True, False)(
        h, agg, deg, W_self_3, W_neigh_3, b_3.reshape(1, 768), Wn4)

    aggz = _sc_agg_split(128)(z, src, dst)
    return _tc_final()(h, aggz, deg, W_self_4, b_4.reshape(1, 47))


# speedup vs baseline: 2.2312x; 1.5791x over previous
import functools

import jax
import jax.numpy as jnp
from jax import lax
from jax.experimental import pallas as pl
from jax.experimental.pallas import tpu as pltpu
from jax.experimental.pallas import tpu_sc as plsc

N = 10000
E = 320000
NC = 2
NS = 16
BLK = 128
G = 8
NBLK = E // BLK
PBLK = 2560
EP = PBLK * BLK
NA = N + 8
GRP = 16
NG = N // GRP
NGJ = -(-NG // NS)
DEGW = 128
F32 = jnp.float32


def _mesh():
    return plsc.VectorSubcoreMesh(
        core_axis_name="c", subcore_axis_name="s",
        num_cores=NC, num_subcores=NS)


def _row_groups(sid, fn):
    def g(j, _):
        grp = sid + NS * j

        @pl.when(grp < NG)
        def _():
            fn(pl.multiple_of(grp * GRP, GRP))
        return 0
    lax.fori_loop(0, NGJ, g, 0)


def _fill_rows(ref, nrows, width, value):
    def outer(i, _):
        def inner(j, _):
            ref[i, pl.ds(j * 16, 16)] = jnp.full((16,), value, F32)
            return 0
        return lax.fori_loop(0, width // 16, inner, 0)
    lax.fori_loop(0, nrows, outer, 0)


def _edge_pairs(xc, src, dst, sva, svb, dva, dvb, ra, rb, acc, gsem,
                stride_base, npairs):
    def load_idx(blk, sv, dv):
        off = pl.multiple_of(blk * BLK, BLK)
        pltpu.sync_copy(src.at[pl.ds(off, BLK)], sv)
        pltpu.sync_copy(dst.at[pl.ds(off, BLK)], dv)

    def drain(r):
        pltpu.make_async_copy(xc.at[pl.ds(0, BLK)], r, gsem).wait()

    load_idx(stride_base(0), sva, dva)
    pltpu.async_copy(xc.at[sva], ra, gsem)

    def pair(q, _):
        va = stride_base(2 * q) < NBLK
        vb = stride_base(2 * q + 1) < NBLK
        va2 = stride_base(2 * q + 2) < NBLK

        @pl.when(vb)
        def _():
            load_idx(stride_base(2 * q + 1), svb, dvb)

        @pl.when(va)
        def _():
            drain(ra)

        @pl.when(vb)
        def _():
            pltpu.async_copy(xc.at[svb], rb, gsem)

        @pl.when(va)
        def _():
            pltpu.sync_copy(ra, acc.at[dva], add=True)

        @pl.when(va2)
        def _():
            load_idx(stride_base(2 * q + 2), sva, dva)

        @pl.when(vb)
        def _():
            drain(rb)

        @pl.when(va2)
        def _():
            pltpu.async_copy(xc.at[sva], ra, gsem)

        @pl.when(vb)
        def _():
            pltpu.sync_copy(rb, acc.at[dvb], add=True)
        return 0
    lax.fori_loop(0, npairs, pair, 0)


@functools.lru_cache(maxsize=None)
def _sc_agg_full(C, W):
    CPC = -(-C // NC)
    NJ = -(-NBLK // NS)
    scratch = [
        pltpu.VMEM((BLK,), jnp.int32),
        pltpu.VMEM((BLK,), jnp.int32),
        pltpu.VMEM((BLK,), jnp.int32),
        pltpu.VMEM((BLK,), jnp.int32),
        pltpu.VMEM((BLK, W), F32),
        pltpu.VMEM((BLK, W), F32),
        pltpu.VMEM((GRP, W), F32),
        pltpu.VMEM_SHARED((NA, W), F32),
        pltpu.SemaphoreType.DMA,
    ]

    def body(xt, src, dst, out, sva, svb, dva, dvb, ra, rb, zrow_v, acc, gsem):
        cid = lax.axis_index("c")
        sid = lax.axis_index("s")
        _fill_rows(zrow_v, GRP, W, 0.0)

        def process_chunk(c):
            xc = xt.at[c]
            _row_groups(sid, lambda off: pltpu.sync_copy(
                zrow_v, acc.at[pl.ds(off, GRP)]))
            plsc.subcore_barrier()
            _edge_pairs(xc, src, dst, sva, svb, dva, dvb, ra, rb, acc,
                        gsem, lambda j: sid + NS * j, -(-NJ // 2))
            plsc.subcore_barrier()
            _row_groups(sid, lambda off: pltpu.sync_copy(
                acc.at[pl.ds(off, GRP)], out.at[c, pl.ds(off, GRP)]))

        for cp in range(NC):
            @pl.when(cid == cp)
            def _(cp=cp):
                for k in range(CPC):
                    c = k * NC + cp
                    if c < C:
                        process_chunk(c)

    return pl.kernel(
        body,
        out_type=jax.ShapeDtypeStruct((C, N, W), F32),
        mesh=_mesh(),
        scratch_types=scratch,
    )


@functools.lru_cache(maxsize=None)
def _sc_agg_split(W):
    NJ = -(-NBLK // (NC * NS))
    scratch = [
        pltpu.VMEM((BLK,), jnp.int32),
        pltpu.VMEM((BLK,), jnp.int32),
        pltpu.VMEM((BLK,), jnp.int32),
        pltpu.VMEM((BLK,), jnp.int32),
        pltpu.VMEM((BLK, W), F32),
        pltpu.VMEM((BLK, W), F32),
        pltpu.VMEM((GRP, W), F32),
        pltpu.VMEM_SHARED((NA, W), F32),
        pltpu.SemaphoreType.DMA,
    ]

    def body(xt, src, dst, out, sva, svb, dva, dvb, ra, rb, zrow_v, acc, gsem):
        cid = lax.axis_index("c")
        sid = lax.axis_index("s")
        _fill_rows(zrow_v, GRP, W, 0.0)

        for cp in range(NC):
            @pl.when(cid == cp)
            def _(cp=cp):
                _row_groups(sid, lambda off: pltpu.sync_copy(
                    zrow_v, acc.at[pl.ds(off, GRP)]))
                plsc.subcore_barrier()
                _edge_pairs(xt, src, dst, sva, svb, dva, dvb, ra, rb, acc,
                            gsem,
                            lambda j: cp * NS + sid + NC * NS * j,
                            -(-NJ // 2))
                plsc.subcore_barrier()
                _row_groups(sid, lambda off: pltpu.sync_copy(
                    acc.at[pl.ds(off, GRP)], out.at[cp, pl.ds(off, GRP)]))

    return pl.kernel(
        body,
        out_type=jax.ShapeDtypeStruct((NC, N, W), F32),
        mesh=_mesh(),
        scratch_types=scratch,
    )


@functools.lru_cache(maxsize=None)
def _sc_deg():
    TOT = PBLK // (NC * NS)
    MG = TOT // G
    PAIRS = MG // 2
    scratch = [
        pltpu.VMEM((2 * G, BLK), jnp.int32),
        pltpu.VMEM((BLK, DEGW), F32),
        pltpu.VMEM((GRP, DEGW), F32),
        pltpu.VMEM_SHARED((NA, DEGW), F32),
        pltpu.SemaphoreType.DMA,
    ]

    def body(dst2, out, dstm, ones_v, zdeg_v, degacc, isem):
        cid = lax.axis_index("c")
        sid = lax.axis_index("s")
        _fill_rows(ones_v, BLK, DEGW, 1.0)
        _fill_rows(zdeg_v, GRP, DEGW, 0.0)

        def span_scatter_ones(b0):
            def idx_load(m, parity, sync):
                bo = b0 + m * G
                if sync:
                    pltpu.sync_copy(dst2.at[pl.ds(bo, G)],
                                    dstm.at[pl.ds(parity * G, G)])
                else:
                    pltpu.async_copy(dst2.at[pl.ds(bo, G)],
                                     dstm.at[pl.ds(parity * G, G)], isem)

            def idx_drain(parity):
                pltpu.make_async_copy(
                    dst2.at[pl.ds(0, G)],
                    dstm.at[pl.ds(parity * G, G)], isem).wait()

            def steps(parity):
                for i in range(G):
                    pltpu.sync_copy(
                        ones_v, degacc.at[dstm.at[parity * G + i]], add=True)

            idx_load(0, 0, sync=True)

            def pair(q, _):
                m0 = 2 * q
                last = q == PAIRS - 1
                idx_load(m0 + 1, 1, sync=False)
                steps(0)
                idx_drain(1)

                @pl.when(jnp.logical_not(last))
                def _():
                    idx_load(m0 + 2, 0, sync=False)
                steps(1)

                @pl.when(jnp.logical_not(last))
                def _():
                    idx_drain(0)
                return 0
            lax.fori_loop(0, PAIRS, pair, 0)

        for cp in range(NC):
            @pl.when(cid == cp)
            def _(cp=cp):
                _row_groups(sid, lambda off: pltpu.sync_copy(
                    zdeg_v, degacc.at[pl.ds(off, GRP)]))
                plsc.subcore_barrier()
                span_scatter_ones((cp * NS + sid) * TOT)
                plsc.subcore_barrier()
                _row_groups(sid, lambda off: pltpu.sync_copy(
                    degacc.at[pl.ds(off, GRP)], out.at[cp, pl.ds(off, GRP)]))

    return pl.kernel(
        body,
        out_type=jax.ShapeDtypeStruct((NC, N, DEGW), F32),
        mesh=_mesh(),
        scratch_types=scratch,
    )


@functools.lru_cache(maxsize=None)
def _tc_sage(C_in, dout, act, fuse_z, split_agg, BN=1000):
    C_out = dout // 128
    din = C_in * 128
    CA = NC if split_agg else C_in
    grid = (N // BN,)
    in_specs = [
        pl.BlockSpec((C_in, BN, 128), lambda i: (0, i, 0)),
        pl.BlockSpec((CA, BN, 128), lambda i: (0, i, 0)),
        pl.BlockSpec((NC, BN, DEGW), lambda i: (0, i, 0)),
        pl.BlockSpec((din, dout), lambda i: (0, 0)),
        pl.BlockSpec((din, dout), lambda i: (0, 0)),
        pl.BlockSpec((1, dout), lambda i: (0, 0)),
    ]
    out_shape = [jax.ShapeDtypeStruct((C_out, N, 128), F32)]
    out_specs = [pl.BlockSpec((C_out, BN, 128), lambda i: (0, i, 0))]
    if fuse_z:
        in_specs.append(pl.BlockSpec((dout, 128), lambda i: (0, 0)))
        out_shape.append(jax.ShapeDtypeStruct((N, 128), F32))
        out_specs.append(pl.BlockSpec((BN, 128), lambda i: (i, 0)))

    def body(h_ref, agg_ref, deg_ref, Ws_ref, Wn_ref, b_ref, *rest):
        if fuse_z:
            Wz_ref, out_ref, z_ref = rest
        else:
            (out_ref,) = rest
        deg = (deg_ref[0] + deg_ref[1])[:, 0:1]
        inv = 1.0 / jnp.maximum(deg, 1.0)
        acc = jnp.zeros((BN, dout), F32) + b_ref[...]
        for c in range(C_in):
            acc += jnp.dot(h_ref[c], Ws_ref[pl.ds(c * 128, 128), :],
                           preferred_element_type=F32)
            if not split_agg:
                acc += jnp.dot(agg_ref[c] * inv,
                               Wn_ref[pl.ds(c * 128, 128), :],
                               preferred_element_type=F32)
        if split_agg:
            a = (agg_ref[0] + agg_ref[1]) * inv
            acc += jnp.dot(a, Wn_ref[...], preferred_element_type=F32)
        if act:
            acc = jnp.maximum(acc, 0.0)
        for co in range(C_out):
            out_ref[co] = acc[:, co * 128:(co + 1) * 128]
        if fuse_z:
            z_ref[...] = jnp.dot(acc, Wz_ref[...], preferred_element_type=F32)

    return pl.pallas_call(
        body, grid=grid, in_specs=in_specs,
        out_specs=out_specs, out_shape=out_shape)


@functools.lru_cache(maxsize=None)
def _tc_final(dout=47, BN=1000):
    grid = (N // BN,)
    in_specs = [
        pl.BlockSpec((6, BN, 128), lambda i: (0, i, 0)),
        pl.BlockSpec((NC, BN, 128), lambda i: (0, i, 0)),
        pl.BlockSpec((NC, BN, DEGW), lambda i: (0, i, 0)),
        pl.BlockSpec((768, dout), lambda i: (0, 0)),
        pl.BlockSpec((1, dout), lambda i: (0, 0)),
    ]

    def body(h_ref, aggz_ref, deg_ref, Ws_ref, b_ref, out_ref):
        deg = (deg_ref[0] + deg_ref[1])[:, 0:1]
        inv = 1.0 / jnp.maximum(deg, 1.0)
        acc = jnp.zeros((BN, dout), F32) + b_ref[...]
        for c in range(6):
            acc += jnp.dot(h_ref[c], Ws_ref[pl.ds(c * 128, 128), :],
                           preferred_element_type=F32)
        az = aggz_ref[0] + aggz_ref[1]
        out_ref[...] = acc + az[:, 0:dout] * inv

    return pl.pallas_call(
        body, grid=grid, in_specs=in_specs,
        out_specs=pl.BlockSpec((BN, dout), lambda i: (i, 0)),
        out_shape=jax.ShapeDtypeStruct((N, dout), F32))


def kernel(x, edge_index,
           W_self_0, W_neigh_0, b_0, W_self_1, W_neigh_1, b_1,
           W_self_2, W_neigh_2, b_2, W_self_3, W_neigh_3, b_3,
           W_self_4, W_neigh_4, b_4):
    src = edge_index[0]
    dst = edge_index[1]
    pad = EP - E
    dst2 = jnp.concatenate(
        [dst, jnp.full((pad,), N, jnp.int32)]).reshape(PBLK, BLK)

    deg = _sc_deg()(dst2)
    agg = _sc_agg_split(128)(x, src, dst)
    h = _tc_sage(1, 768, True, False, True)(
        x.reshape(1, N, 128), agg, deg, W_self_0, W_neigh_0,
        b_0.reshape(1, 768))[0]

    for Ws, Wn, b in ((W_self_1, W_neigh_1, b_1), (W_self_2, W_neigh_2, b_2)):
        agg = _sc_agg_full(6, 128)(h, src, dst)
        h = _tc_sage(6, 768, True, False, False)(
            h, agg, deg, Ws, Wn, b.reshape(1, 768))[0]

    agg = _sc_agg_full(6, 128)(h, src, dst)
    Wn4 = jnp.pad(W_neigh_4, ((0, 0), (0, 128 - 47)))
    h, z = _tc_sage(6, 768, True, True, False)(
        h, agg, deg, W_self_3, W_neigh_3, b_3.reshape(1, 768), Wn4)

    aggz = _sc_agg_split(128)(z, src, dst)
    return _tc_final()(h, aggz, deg, W_self_4, b_4.reshape(1, 47))
